# Initial kernel scaffold; baseline (speedup 1.0000x reference)
import jax
import jax.numpy as jnp
from jax.experimental import pallas as pl
from jax.experimental.pallas import tpu as pltpu

N = 512
C = 512
K_SEL = 256
BLK = 16
GRID = N // BLK


def _soft(logits, u):
    z = logits + (jnp.log(u) - jnp.log1p(-u))
    return jax.nn.sigmoid(z)


def _kth_smallest(keys, axis):
    shape = (1, N) if axis == 0 else (N, 1)
    prefix = jnp.zeros(shape, jnp.int32)
    want = jnp.full(shape, K_SEL, jnp.int32)
    for b in range(30, -1, -1):
        high_match = (keys >> (b + 1)) == (prefix >> (b + 1))
        bit_is0 = ((keys >> b) & 1) == 0
        cnt0 = jnp.sum(
            jnp.where(high_match & bit_is0, 1, 0).astype(jnp.int32),
            axis=axis, keepdims=True)
        take1 = want > cnt0
        prefix = jnp.where(take1, prefix | (1 << b), prefix)
        want = jnp.where(take1, want - cnt0, want)
    return prefix


def _body(w_ref, b_ref, xab_ref, xba_ref, uab_ref, uba_ref, y_ref,
          sa_ref, sb_ref):
    g = pl.program_id(0)
    w = w_ref[...].reshape(1, 1, C)
    bias = b_ref[0]
    la = jnp.sum(xab_ref[...] * w, axis=-1) + bias
    lb = jnp.sum(xba_ref[...] * w, axis=-1) + bias
    sa_ref[pl.ds(g * BLK, BLK), :] = _soft(la, uab_ref[...])
    sb_ref[pl.ds(g * BLK, BLK), :] = _soft(lb, uba_ref[...])

    @pl.when(g == GRID - 1)
    def _():
        sa = sa_ref[...]
        sb = sb_ref[...]
        ka = jax.lax.bitcast_convert_type(sa, jnp.int32)
        kb = jax.lax.bitcast_convert_type(sb, jnp.int32)
        thr_a = jax.lax.bitcast_convert_type(_kth_smallest(ka, 0), jnp.float32)
        thr_b = jax.lax.bitcast_convert_type(_kth_smallest(kb, 1), jnp.float32)
        ha = (sa >= thr_a).astype(jnp.float32)
        hb = (sb >= thr_b).astype(jnp.float32)
        ysum = ha + hb
        y_ref[...] = jnp.where(ysum == 2.0, 1.0, ysum)


def kernel(xab, xba_t, W, b, u_ab, u_ba):
    u2a = u_ab.reshape(N, N)
    u2b = u_ba.reshape(N, N)
    y = pl.pallas_call(
        _body,
        grid=(GRID,),
        in_specs=[
            pl.BlockSpec((1, C), lambda g: (0, 0)),
            pl.BlockSpec(memory_space=pltpu.SMEM),
            pl.BlockSpec((BLK, N, C), lambda g: (g, 0, 0)),
            pl.BlockSpec((BLK, N, C), lambda g: (g, 0, 0)),
            pl.BlockSpec((BLK, N), lambda g: (g, 0)),
            pl.BlockSpec((BLK, N), lambda g: (g, 0)),
        ],
        out_specs=pl.BlockSpec((N, N), lambda g: (0, 0)),
        out_shape=jax.ShapeDtypeStruct((N, N), jnp.float32),
        scratch_shapes=[
            pltpu.VMEM((N, N), jnp.float32),
            pltpu.VMEM((N, N), jnp.float32),
        ],
    )(W, b, xab, xba_t, u2a, u2b)
    return y.reshape(N, N, 1)


# speedup vs baseline: 10.9252x; 10.9252x over previous
import jax
import jax.numpy as jnp
from jax.experimental import pallas as pl
from jax.experimental.pallas import tpu as pltpu

N = 512
C = 512
K_SEL = 256
BLK = 8
GRID = N // BLK


def _soft(logits, u):
    z = logits + (jnp.log(u) - jnp.log1p(-u))
    return jax.nn.sigmoid(z)


def _kth_smallest(keys, axis):
    shape = (1, N) if axis == 0 else (N, 1)

    def step(i, carry):
        prefix, want = carry
        b = 30 - i
        high_match = (keys >> (b + 1)) == (prefix >> (b + 1))
        bit_is0 = ((keys >> b) & 1) == 0
        cnt0 = jnp.sum(
            jnp.where(high_match & bit_is0, 1, 0).astype(jnp.int32),
            axis=axis, keepdims=True)
        take1 = want > cnt0
        prefix = jnp.where(take1, prefix | (1 << b), prefix)
        want = jnp.where(take1, want - cnt0, want)
        return prefix, want

    prefix, _ = jax.lax.fori_loop(
        0, 31, step,
        (jnp.zeros(shape, jnp.int32), jnp.full(shape, K_SEL, jnp.int32)))
    return prefix


def _body(w_ref, b_ref, xab_ref, xba_ref, uab_ref, uba_ref, y_ref,
          sa_ref, sb_ref):
    g = pl.program_id(0)
    w = w_ref[...]
    bias = b_ref[0]
    xa2 = xab_ref[...].reshape(BLK * N, C)
    xb2 = xba_ref[...].reshape(BLK * N, C)
    la = jnp.dot(xa2, w).reshape(BLK, N) + bias
    lb = jnp.dot(xb2, w).reshape(BLK, N) + bias
    sa_ref[pl.ds(g * BLK, BLK), :] = _soft(la, uab_ref[...])
    sb_ref[pl.ds(g * BLK, BLK), :] = _soft(lb, uba_ref[...])

    @pl.when(g == GRID - 1)
    def _():
        sa = sa_ref[...]
        sb = sb_ref[...]
        ka = jax.lax.bitcast_convert_type(sa, jnp.int32)
        kb = jax.lax.bitcast_convert_type(sb, jnp.int32)
        thr_a = jax.lax.bitcast_convert_type(_kth_smallest(ka, 0), jnp.float32)
        thr_b = jax.lax.bitcast_convert_type(_kth_smallest(kb, 1), jnp.float32)
        ha = (sa >= thr_a).astype(jnp.float32)
        hb = (sb >= thr_b).astype(jnp.float32)
        ysum = ha + hb
        y_ref[...] = jnp.where(ysum == 2.0, 1.0, ysum)


def kernel(xab, xba_t, W, b, u_ab, u_ba):
    u2a = u_ab.reshape(N, N)
    u2b = u_ba.reshape(N, N)
    y = pl.pallas_call(
        _body,
        grid=(GRID,),
        in_specs=[
            pl.BlockSpec((C, 1), lambda g: (0, 0)),
            pl.BlockSpec(memory_space=pltpu.SMEM),
            pl.BlockSpec((BLK, N, C), lambda g: (g, 0, 0)),
            pl.BlockSpec((BLK, N, C), lambda g: (g, 0, 0)),
            pl.BlockSpec((BLK, N), lambda g: (g, 0)),
            pl.BlockSpec((BLK, N), lambda g: (g, 0)),
        ],
        out_specs=pl.BlockSpec((N, N), lambda g: (0, 0)),
        out_shape=jax.ShapeDtypeStruct((N, N), jnp.float32),
        scratch_shapes=[
            pltpu.VMEM((N, N), jnp.float32),
            pltpu.VMEM((N, N), jnp.float32),
        ],
    )(W.reshape(C, 1), b, xab, xba_t, u2a, u2b)
    return y.reshape(N, N, 1)
